# Initial kernel scaffold; baseline (speedup 1.0000x reference)
import functools

import jax
import jax.numpy as jnp
from jax import lax
from jax.experimental import pallas as pl
from jax.experimental.pallas import tpu as pltpu
from jax.experimental.pallas import tpu_sc as plsc

NUM_CLASSES = 64
D = 128
N = 320000
NC, NS = 2, 16
NW = NC * NS
BLK = 128
NB = N // BLK
ITERS = (NB + NW - 1) // NW
CW = 16


def _sc_body(feat_hbm, lab_hbm, ones_hbm, zsum_hbm, zcnt_hbm,
             sums_out, cnts_out, fblk, lblk, ones_v, acc_sh, cnt_sh):
  cid = lax.axis_index("c")
  sid = lax.axis_index("s")
  wid = sid * NC + cid

  @pl.when(sid == 0)
  def _():
    pltpu.sync_copy(zsum_hbm, acc_sh)
    pltpu.sync_copy(zcnt_hbm, cnt_sh)

  pltpu.sync_copy(ones_hbm, ones_v)
  plsc.subcore_barrier()

  @pl.loop(0, ITERS)
  def _(j):
    bid = j * NW + wid

    @pl.when(bid < NB)
    def _():
      pltpu.sync_copy(lab_hbm.at[bid], lblk)
      pltpu.sync_copy(feat_hbm.at[bid], fblk)
      pltpu.sync_copy(fblk, acc_sh.at[lblk], add=True)
      pltpu.sync_copy(ones_v, cnt_sh.at[lblk], add=True)

  plsc.subcore_barrier()

  @pl.when(sid == 0)
  def _():
    pltpu.sync_copy(acc_sh, sums_out.at[cid])
    pltpu.sync_copy(cnt_sh, cnts_out.at[cid])


_sc_segment_sums = functools.partial(
    pl.kernel,
    out_type=(
        jax.ShapeDtypeStruct((NC, NUM_CLASSES, D), jnp.float32),
        jax.ShapeDtypeStruct((NC, NUM_CLASSES, CW), jnp.float32),
    ),
    mesh=plsc.VectorSubcoreMesh(core_axis_name="c", subcore_axis_name="s",
                                num_cores=NC, num_subcores=NS),
    scratch_types=[
        pltpu.VMEM((BLK, D), jnp.float32),
        pltpu.VMEM((BLK,), jnp.int32),
        pltpu.VMEM((BLK, CW), jnp.float32),
        pltpu.VMEM_SHARED((NUM_CLASSES, D), jnp.float32),
        pltpu.VMEM_SHARED((NUM_CLASSES, CW), jnp.float32),
    ],
)(_sc_body)


def _combine_body(sums_ref, cnts_ref, out_ref):
  s = sums_ref[0] + sums_ref[1]
  c = cnts_ref[0] + cnts_ref[1]
  out_ref[...] = s / c[:, 0:1]


def kernel(support_features, support_labels):
  feat = support_features.reshape(NB, BLK, D)
  lab = support_labels.astype(jnp.int32).reshape(NB, BLK)
  ones = jnp.ones((BLK, CW), jnp.float32)
  zsum = jnp.zeros((NUM_CLASSES, D), jnp.float32)
  zcnt = jnp.zeros((NUM_CLASSES, CW), jnp.float32)

  sums, cnts = _sc_segment_sums(feat, lab, ones, zsum, zcnt)

  return pl.pallas_call(
      _combine_body,
      out_shape=jax.ShapeDtypeStruct((NUM_CLASSES, D), jnp.float32),
  )(sums, cnts)


# speedup vs baseline: 4.7275x; 4.7275x over previous
import functools

import jax
import jax.numpy as jnp
from jax import lax
from jax.experimental import pallas as pl
from jax.experimental.pallas import tpu as pltpu
from jax.experimental.pallas import tpu_sc as plsc

NUM_CLASSES = 64
D = 128
N = 320000
NC, NS = 2, 16
NW = NC * NS
BLK = 128
NB = N // BLK
ITERS = (NB + NW - 1) // NW


def _sc_body(feat_hbm, lab_hbm, ones_hbm, zsum_hbm, sums_out, cnts_out,
             fblk, lblk, ones_v, acc_sh, cnt_sh):
  cid = lax.axis_index("c")
  sid = lax.axis_index("s")
  wid = sid * NC + cid

  @pl.when(sid == 0)
  def _():
    pltpu.sync_copy(zsum_hbm, acc_sh)
    pltpu.sync_copy(zsum_hbm, cnt_sh)

  pltpu.sync_copy(ones_hbm, ones_v)
  plsc.subcore_barrier()

  @pl.loop(0, ITERS)
  def _(j):
    bid = j * NW + wid

    @pl.when(bid < NB)
    def _():
      pltpu.sync_copy(lab_hbm.at[bid], lblk)
      pltpu.sync_copy(feat_hbm.at[bid], fblk)
      pltpu.sync_copy(fblk, acc_sh.at[lblk], add=True)
      pltpu.sync_copy(ones_v, cnt_sh.at[lblk], add=True)

  plsc.subcore_barrier()

  @pl.when(sid == 0)
  def _():
    pltpu.sync_copy(acc_sh, sums_out.at[cid])
    pltpu.sync_copy(cnt_sh, cnts_out.at[cid])


_sc_segment_sums = functools.partial(
    pl.kernel,
    out_type=(
        jax.ShapeDtypeStruct((NC, NUM_CLASSES, D), jnp.float32),
        jax.ShapeDtypeStruct((NC, NUM_CLASSES, D), jnp.float32),
    ),
    mesh=plsc.VectorSubcoreMesh(core_axis_name="c", subcore_axis_name="s",
                                num_cores=NC, num_subcores=NS),
    scratch_types=[
        pltpu.VMEM((BLK, D), jnp.float32),
        pltpu.VMEM((BLK,), jnp.int32),
        pltpu.VMEM((BLK, D), jnp.float32),
        pltpu.VMEM_SHARED((NUM_CLASSES, D), jnp.float32),
        pltpu.VMEM_SHARED((NUM_CLASSES, D), jnp.float32),
    ],
)(_sc_body)


def _combine_body(sums_ref, cnts_ref, out_ref):
  s = sums_ref[0] + sums_ref[1]
  c = cnts_ref[0] + cnts_ref[1]
  out_ref[...] = s / c[:, 0:1]


def kernel(support_features, support_labels):
  feat = support_features.reshape(NB, BLK, D)
  lab = support_labels.astype(jnp.int32).reshape(NB, BLK)
  zsum = jnp.zeros((NUM_CLASSES, D), jnp.float32)
  ones = jnp.ones((BLK, D), jnp.float32)

  sums, cnts = _sc_segment_sums(feat, lab, ones, zsum)

  return pl.pallas_call(
      _combine_body,
      out_shape=jax.ShapeDtypeStruct((NUM_CLASSES, D), jnp.float32),
  )(sums, cnts)
